# baseline (speedup 1.0000x reference)
import functools

import jax
import jax.numpy as jnp
from jax import lax
from jax.experimental import pallas as pl
from jax.experimental.pallas import tpu as pltpu
from jax.experimental.pallas import tpu_sc as plsc

BATCH = 512
NUM_TTA = 64
NUM_CLASSES = 1000
KEPT_VIEWS = 6
EPS = 1e-08

SAMP_A = 8
GRID_A = BATCH // SAMP_A

NC = 2
NS = 16
L = 16
NW = NC * NS
SAMP_SC = 16
HALF = BATCH // 2
ACT_W = HALF // SAMP_SC
GRID_H = HALF // SAMP_A
TAB = 80


def _stage_a(x_ref, ent_ref, vote_ref, tab_ref):
    p = x_ref[...]
    safe = jnp.maximum(p, jnp.float32(1e-37))
    ent_ref[...] = -jnp.sum(p * jnp.log(safe), axis=2)
    m = jnp.max(p, axis=2)
    lane = lax.broadcasted_iota(jnp.int32, p.shape, 2)
    vote_ref[...] = jnp.min(
        jnp.where(p == m[:, :, None], lane, NUM_CLASSES), axis=2)

    @pl.when(pl.program_id(0) == 0)
    def _():
        c = lax.broadcasted_iota(jnp.int32, (1, 128), 1).astype(jnp.float32)
        tab_ref[...] = jnp.log(c * (1.0 / NUM_TTA) + EPS)


def _sc_body(ent_sm, votes_sm, logtab, out,
             ent_sv, votes_sv, ent_v, votes_v, rank_v, sv_f, cnt_f,
             rows_v, tab_v):
    wid = lax.axis_index("s") * NC + lax.axis_index("c")

    @pl.when(wid < ACT_W)
    def _active():
        _sc_work(ent_sm, votes_sm, logtab, out, ent_sv, votes_sv, ent_v,
                 votes_v, rank_v, sv_f, cnt_f, rows_v, tab_v,
                 wid * SAMP_SC)


def _sc_work(ent_sm, votes_sm, logtab, out, ent_sv, votes_sv, ent_v,
             votes_v, rank_v, sv_f, cnt_f, rows_v, tab_v, base):
    i32 = jnp.int32
    pltpu.sync_copy(ent_sm.at[pl.ds(base, SAMP_SC)], ent_sv)
    pltpu.sync_copy(votes_sm.at[pl.ds(base, SAMP_SC)], votes_sv)
    pltpu.sync_copy(logtab, tab_v)

    lane = lax.iota(i32, L)
    zeros = jnp.zeros((L,), i32)

    def tbody(i, _):
        vcol = jnp.full((L,), i, i32)
        ent_v[i, :] = plsc.load_gather(ent_sv, [lane, vcol])
        votes_v[i, :] = plsc.load_gather(votes_sv, [lane, vcol])
        return 0
    lax.fori_loop(0, NUM_TTA, tbody, 0, unroll=8)

    def zbody(r, _):
        cnt_f[pl.ds(r * L, L)] = zeros
        return 0
    lax.fori_loop(0, NUM_CLASSES, zbody, 0, unroll=8)

    def rbody(ib, ssum):
        e_a = ent_v[2 * ib, :]
        e_b = ent_v[2 * ib + 1, :]

        def jall(j, rs):
            ra, rb = rs
            e_j = ent_v[j, :]
            ra = ra + jnp.where(e_j < e_a, 1, 0)
            rb = rb + jnp.where(e_j < e_b, 1, 0)
            return ra, rb

        ra, rb = lax.fori_loop(0, NUM_TTA, jall, (zeros, zeros), unroll=8)
        rank_v[2 * ib, :] = ra
        rank_v[2 * ib + 1, :] = rb
        return ssum + ra + rb
    ssum = lax.fori_loop(0, NUM_TTA // 2, rbody, zeros)

    tie_any = jnp.max(jnp.where(ssum != NUM_TTA * (NUM_TTA - 1) // 2, 1, 0))

    @pl.when(tie_any > 0)
    def _():
        def fix(i, _):
            e_i = ent_v[i, :]

            def jeq(j, r):
                return r + jnp.where(ent_v[j, :] == e_i, 1, 0)

            extra = lax.fori_loop(0, i, jeq, zeros)
            rank_v[i, :] = rank_v[i, :] + extra
            return 0
        lax.fori_loop(0, NUM_TTA, fix, 0)

    def sbody(i, _):
        plsc.store_scatter(sv_f, [rank_v[i, :] * L + lane], votes_v[i, :])
        return 0
    lax.fori_loop(0, NUM_TTA, sbody, 0, unroll=8)

    def vbody(t, carry):
        mx, nmx, active = carry
        v_t = sv_f[pl.ds(t * L, L)]
        idx = v_t * L + lane
        new = plsc.load_gather(cnt_f, [idx]) + 1
        plsc.store_scatter(cnt_f, [idx], new, mask=active)
        upd = active & (new > mx)
        tie = active & (new == mx)
        mx = jnp.where(upd, new, mx)
        nmx = jnp.where(upd, 1, jnp.where(tie, nmx + 1, nmx))
        stop = (t + 1 >= KEPT_VIEWS) & (nmx == 1)
        active = active & jnp.logical_not(stop)
        return mx, nmx, active
    lax.fori_loop(0, NUM_TTA, vbody,
                  (zeros, zeros, jnp.ones((L,), jnp.bool_)))

    def srow(s, _):
        def cchunk(k, _):
            c0 = jnp.minimum(k * L, NUM_CLASSES - L)
            cnts = plsc.load_gather(cnt_f, [(lane + c0) * L + s])
            rows_v[s, pl.ds(c0, L)] = plsc.load_gather(tab_v, [cnts])
            return 0
        lax.fori_loop(0, (NUM_CLASSES + L - 1) // L, cchunk, 0, unroll=4)
        return 0
    lax.fori_loop(0, SAMP_SC, srow, 0)
    pltpu.sync_copy(rows_v, out.at[pl.ds(base, SAMP_SC)])


_stage_b_sc = functools.partial(
    pl.kernel,
    mesh=plsc.VectorSubcoreMesh(core_axis_name="c", subcore_axis_name="s"),
    compiler_params=pltpu.CompilerParams(needs_layout_passes=False),
    out_type=jax.ShapeDtypeStruct((HALF, NUM_CLASSES), jnp.float32),
    scratch_types=[
        pltpu.VMEM((SAMP_SC, NUM_TTA), jnp.float32),
        pltpu.VMEM((SAMP_SC, NUM_TTA), jnp.int32),
        pltpu.VMEM((NUM_TTA, SAMP_SC), jnp.float32),
        pltpu.VMEM((NUM_TTA, SAMP_SC), jnp.int32),
        pltpu.VMEM((NUM_TTA, SAMP_SC), jnp.int32),
        pltpu.VMEM((NUM_TTA * L,), jnp.int32),
        pltpu.VMEM((NUM_CLASSES * L,), jnp.int32),
        pltpu.VMEM((SAMP_SC, NUM_CLASSES), jnp.float32),
        pltpu.VMEM((TAB,), jnp.float32),
    ],
)(_sc_body)


@jax.jit
def kernel(x):
    x4 = x.reshape(2, HALF, NUM_TTA, NUM_CLASSES)

    def run_half(xh):
        ent, votes, tab = pl.pallas_call(
            _stage_a,
            grid=(GRID_H,),
            in_specs=[
                pl.BlockSpec((SAMP_A, NUM_TTA, NUM_CLASSES),
                             lambda i: (i, 0, 0)),
            ],
            out_specs=[
                pl.BlockSpec((SAMP_A, NUM_TTA), lambda i: (i, 0)),
                pl.BlockSpec((SAMP_A, NUM_TTA), lambda i: (i, 0)),
                pl.BlockSpec((1, 128), lambda i: (0, 0)),
            ],
            out_shape=[
                jax.ShapeDtypeStruct((HALF, NUM_TTA), jnp.float32),
                jax.ShapeDtypeStruct((HALF, NUM_TTA), jnp.int32),
                jax.ShapeDtypeStruct((1, 128), jnp.float32),
            ],
        )(xh)
        return ent, votes, tab.reshape(128)[:TAB]

    ent_a, votes_a, logtab = run_half(x4[0])
    ent_b, votes_b, _ = run_half(x4[1])
    out_a = _stage_b_sc(ent_a, votes_a, logtab)
    out_b = _stage_b_sc(ent_b, votes_b, logtab)
    return jnp.concatenate([out_a, out_b], axis=0)


# speedup vs baseline: 1.4842x; 1.4842x over previous
import functools

import jax
import jax.numpy as jnp
from jax import lax
from jax.experimental import pallas as pl
from jax.experimental.pallas import tpu as pltpu
from jax.experimental.pallas import tpu_sc as plsc

BATCH = 512
NUM_TTA = 64
NUM_CLASSES = 1000
KEPT_VIEWS = 6
EPS = 1e-08

SAMP_A = 8
GRID_A = BATCH // SAMP_A

NC = 2
NS = 16
L = 16
NW = NC * NS
SAMP_SC = BATCH // NW
TAB = 80


def _stage_a(x_ref, ent_ref, vote_ref, tab_ref):
    p = x_ref[...]
    safe = jnp.maximum(p, jnp.float32(1e-37))
    ent_ref[...] = -jnp.sum(p * jnp.log(safe), axis=2)
    m = jnp.max(p, axis=2)
    lane = lax.broadcasted_iota(jnp.int32, p.shape, 2)
    vote_ref[...] = jnp.min(
        jnp.where(p == m[:, :, None], lane, NUM_CLASSES), axis=2)

    @pl.when(pl.program_id(0) == 0)
    def _():
        c = lax.broadcasted_iota(jnp.int32, (1, 128), 1).astype(jnp.float32)
        tab_ref[...] = jnp.log(c * (1.0 / NUM_TTA) + EPS)


def _sc_body(ent_sm, votes_sm, logtab, out,
             ent_sv, votes_sv, ent_v, votes_v, rank_v, sv_f, cnt_f,
             rows_v, tab_v):
    i32 = jnp.int32
    wid = lax.axis_index("s") * NC + lax.axis_index("c")
    base = wid * SAMP_SC
    pltpu.sync_copy(ent_sm.at[pl.ds(base, SAMP_SC)], ent_sv)
    pltpu.sync_copy(votes_sm.at[pl.ds(base, SAMP_SC)], votes_sv)
    pltpu.sync_copy(logtab, tab_v)

    lane = lax.iota(i32, L)
    zeros = jnp.zeros((L,), i32)

    def tbody(i, _):
        vcol = jnp.full((L,), i, i32)
        ent_v[i, :] = plsc.load_gather(ent_sv, [lane, vcol])
        votes_v[i, :] = plsc.load_gather(votes_sv, [lane, vcol])
        return 0
    lax.fori_loop(0, NUM_TTA, tbody, 0, unroll=8)

    def zbody(r, _):
        cnt_f[pl.ds(r * L, L)] = zeros
        return 0
    lax.fori_loop(0, NUM_CLASSES, zbody, 0, unroll=8)

    def rbody(ib, ssum):
        e_a = ent_v[2 * ib, :]
        e_b = ent_v[2 * ib + 1, :]

        def jall(j, rs):
            ra, rb = rs
            e_j = ent_v[j, :]
            ra = ra + jnp.where(e_j < e_a, 1, 0)
            rb = rb + jnp.where(e_j < e_b, 1, 0)
            return ra, rb

        ra, rb = lax.fori_loop(0, NUM_TTA, jall, (zeros, zeros), unroll=8)
        rank_v[2 * ib, :] = ra
        rank_v[2 * ib + 1, :] = rb
        return ssum + ra + rb
    ssum = lax.fori_loop(0, NUM_TTA // 2, rbody, zeros)

    tie_any = jnp.max(jnp.where(ssum != NUM_TTA * (NUM_TTA - 1) // 2, 1, 0))

    @pl.when(tie_any > 0)
    def _():
        def fix(i, _):
            e_i = ent_v[i, :]

            def jeq(j, r):
                return r + jnp.where(ent_v[j, :] == e_i, 1, 0)

            extra = lax.fori_loop(0, i, jeq, zeros)
            rank_v[i, :] = rank_v[i, :] + extra
            return 0
        lax.fori_loop(0, NUM_TTA, fix, 0)

    def sbody(i, _):
        plsc.store_scatter(sv_f, [rank_v[i, :] * L + lane], votes_v[i, :])
        return 0
    lax.fori_loop(0, NUM_TTA, sbody, 0, unroll=8)

    def vbody(t, carry):
        mx, nmx, active = carry
        v_t = sv_f[pl.ds(t * L, L)]
        idx = v_t * L + lane
        new = plsc.load_gather(cnt_f, [idx]) + 1
        plsc.store_scatter(cnt_f, [idx], new, mask=active)
        upd = active & (new > mx)
        tie = active & (new == mx)
        mx = jnp.where(upd, new, mx)
        nmx = jnp.where(upd, 1, jnp.where(tie, nmx + 1, nmx))
        stop = (t + 1 >= KEPT_VIEWS) & (nmx == 1)
        active = active & jnp.logical_not(stop)
        return mx, nmx, active
    lax.fori_loop(0, NUM_TTA, vbody,
                  (zeros, zeros, jnp.ones((L,), jnp.bool_)))

    def srow(s, _):
        def cchunk(k, _):
            c0 = jnp.minimum(k * L, NUM_CLASSES - L)
            cnts = plsc.load_gather(cnt_f, [(lane + c0) * L + s])
            rows_v[s, pl.ds(c0, L)] = plsc.load_gather(tab_v, [cnts])
            return 0
        lax.fori_loop(0, (NUM_CLASSES + L - 1) // L, cchunk, 0, unroll=4)
        return 0
    lax.fori_loop(0, SAMP_SC, srow, 0)
    pltpu.sync_copy(rows_v, out.at[pl.ds(base, SAMP_SC)])


_stage_b_sc = functools.partial(
    pl.kernel,
    mesh=plsc.VectorSubcoreMesh(core_axis_name="c", subcore_axis_name="s"),
    compiler_params=pltpu.CompilerParams(needs_layout_passes=False),
    out_type=jax.ShapeDtypeStruct((BATCH, NUM_CLASSES), jnp.float32),
    scratch_types=[
        pltpu.VMEM((SAMP_SC, NUM_TTA), jnp.float32),
        pltpu.VMEM((SAMP_SC, NUM_TTA), jnp.int32),
        pltpu.VMEM((NUM_TTA, SAMP_SC), jnp.float32),
        pltpu.VMEM((NUM_TTA, SAMP_SC), jnp.int32),
        pltpu.VMEM((NUM_TTA, SAMP_SC), jnp.int32),
        pltpu.VMEM((NUM_TTA * L,), jnp.int32),
        pltpu.VMEM((NUM_CLASSES * L,), jnp.int32),
        pltpu.VMEM((SAMP_SC, NUM_CLASSES), jnp.float32),
        pltpu.VMEM((TAB,), jnp.float32),
    ],
)(_sc_body)


@jax.jit
def kernel(x):
    x3 = x.reshape(BATCH, NUM_TTA, NUM_CLASSES)
    ent, votes, tab = pl.pallas_call(
        _stage_a,
        grid=(GRID_A,),
        in_specs=[
            pl.BlockSpec((SAMP_A, NUM_TTA, NUM_CLASSES), lambda i: (i, 0, 0)),
        ],
        out_specs=[
            pl.BlockSpec((SAMP_A, NUM_TTA), lambda i: (i, 0)),
            pl.BlockSpec((SAMP_A, NUM_TTA), lambda i: (i, 0)),
            pl.BlockSpec((1, 128), lambda i: (0, 0)),
        ],
        out_shape=[
            jax.ShapeDtypeStruct((BATCH, NUM_TTA), jnp.float32),
            jax.ShapeDtypeStruct((BATCH, NUM_TTA), jnp.int32),
            jax.ShapeDtypeStruct((1, 128), jnp.float32),
        ],
    )(x3)
    logtab = tab.reshape(128)[:TAB]
    return _stage_b_sc(ent, votes, logtab)
